# baseline (speedup 1.0000x reference)
import functools

import jax
import jax.numpy as jnp
from jax.experimental import pallas as pl
from jax.experimental.pallas import tpu as pltpu
from jax.experimental.pallas import tpu_sc as plsc

_N = 160000
_D = 128
_M = 1024
_B = 8000
_NB = _N // _B

_NW = 32
_GRP = 20
_BATCH = 128
_CHUNK = _GRP * _BATCH
_NPAD = _NW * _CHUNK
_NH = _N // 2
_NBH = _NH // _B
_f32 = jnp.float32
_i32 = jnp.int32


def _tc_body(f0_ref, f1_ref, w0_ref, w1_ref, vals_ref):
    p0 = jax.lax.dot_general(
        f0_ref[0].astype(jnp.bfloat16), w0_ref[...],
        (((1,), (0,)), ((), ())), preferred_element_type=_f32)
    p1 = jax.lax.dot_general(
        f1_ref[0].astype(jnp.bfloat16), w1_ref[...],
        (((1,), (0,)), ((), ())), preferred_element_type=_f32)
    vals_ref[...] = p0 + p1


def _run_tc(feats, w0p, w1p, half):
    off = half * _NBH
    return pl.pallas_call(
        _tc_body,
        grid=(_NBH,),
        in_specs=[
            pl.BlockSpec((1, _B, _D), lambda i: (0, i + off, 0)),
            pl.BlockSpec((1, _B, _D), lambda i: (1, i + off, 0)),
            pl.BlockSpec((_D, 8), lambda i: (0, 0)),
            pl.BlockSpec((_D, 8), lambda i: (0, 0)),
        ],
        out_specs=pl.BlockSpec((_B, 8), lambda i: (i, 0)),
        out_shape=jax.ShapeDtypeStruct((_NPAD, 8), _f32),
    )(feats, feats, w0p, w1p)


def _sc_body(vals_hbm, mol_hbm, db_hbm, zeros_hbm,
             part_hbm, atomen_hbm, ahier_hbm,
             idx_v, vals_v, atomen_v, ahier_v, db_v, acc_sh):
    c = jax.lax.axis_index("c")
    s = jax.lax.axis_index("s")
    wid = s * 2 + c
    base = wid * _CHUNK
    pltpu.sync_copy(mol_hbm.at[wid], idx_v)
    pltpu.sync_copy(vals_hbm.at[pl.ds(base, _CHUNK)], vals_v)
    pltpu.sync_copy(db_hbm, db_v)

    @pl.when(s == 0)
    def _zero():
        pltpu.sync_copy(zeros_hbm, acc_sh)

    dep = db_v[0]
    b1 = db_v[1]
    lanes = jax.lax.iota(_i32, 16)
    col0 = jnp.zeros((16,), _i32)

    def _atom(i, carry):
        rows = i * 16 + lanes
        pe0 = plsc.load_gather(vals_v, [rows, col0]) + dep
        pe1 = plsc.load_gather(vals_v, [rows, col0 + 1]) + b1
        e0s = pe0 * pe0
        e1s = pe1 * pe1
        den = e0s + e1s
        hier = e1s / den
        atomen_v[pl.ds(i * 16, 16)] = pe0 + pe1
        ahier_v[pl.ds(i * 16, 16)] = hier
        plsc.store_scatter(vals_v, [rows, col0], pe0)
        plsc.store_scatter(vals_v, [rows, col0 + 1], pe1)
        plsc.store_scatter(vals_v, [rows, col0 + 2], hier)
        plsc.store_scatter(vals_v, [rows, col0 + 3], e1s)
        plsc.store_scatter(vals_v, [rows, col0 + 4], den)
        return carry

    jax.lax.fori_loop(0, _CHUNK // 16, _atom, 0)
    pltpu.sync_copy(atomen_v, atomen_hbm.at[pl.ds(base, _CHUNK)])
    pltpu.sync_copy(ahier_v, ahier_hbm.at[pl.ds(base, _CHUNK)])
    plsc.subcore_barrier()

    def _grp(g, carry):
        pltpu.sync_copy(vals_v.at[pl.ds(g * _BATCH, _BATCH)],
                        acc_sh.at[idx_v.at[g]], add=True)
        return carry

    jax.lax.fori_loop(0, _GRP, _grp, 0)
    plsc.subcore_barrier()

    @pl.when(s == 0)
    def _flush():
        pltpu.sync_copy(acc_sh.at[pl.ds(0, _M)], part_hbm.at[c])


_sc_segsum = functools.partial(
    pl.kernel,
    out_type=[
        jax.ShapeDtypeStruct((2, _M, 8), _f32),
        jax.ShapeDtypeStruct((_NPAD,), _f32),
        jax.ShapeDtypeStruct((_NPAD,), _f32),
    ],
    mesh=plsc.VectorSubcoreMesh(core_axis_name="c", subcore_axis_name="s"),
    compiler_params=pltpu.CompilerParams(use_tc_tiling_on_sc=False,
                                         needs_layout_passes=False),
    scratch_types=[
        pltpu.VMEM((_GRP, _BATCH), _i32),
        pltpu.VMEM((_CHUNK, 8), _f32),
        pltpu.VMEM((_CHUNK,), _f32),
        pltpu.VMEM((_CHUNK,), _f32),
        pltpu.VMEM((2, 16), _f32),
        pltpu.VMEM_SHARED((_M + 1, 8), _f32),
    ],
)(_sc_body)


def _fin_body(pa_ref, pb_ref, te_ref, p0_ref, p1_ref, th_ref, mh_ref, bh_ref):
    p = (pa_ref[0] + pa_ref[1]) + (pb_ref[0] + pb_ref[1])
    t0 = p[:, 0:1]
    t1 = p[:, 1:2]
    te_ref[...] = t0 + t1
    p0_ref[...] = t0
    p1_ref[...] = t0 + t1
    th_ref[...] = p[:, 2:3]
    mh_ref[...] = p[:, 3:4] / p[:, 4:5]
    bh_ref[...] = (jnp.sum(p[:, 3:4], keepdims=True) /
                   jnp.sum(p[:, 4:5], keepdims=True))


def _run_fin(pa, pb):
    m1 = [jax.ShapeDtypeStruct((_M, 1), _f32)] * 5
    return pl.pallas_call(
        _fin_body,
        out_shape=m1 + [jax.ShapeDtypeStruct((1, 1), _f32)],
    )(pa, pb)


def kernel(all_features, mol_index, n_molecules, W0, W1, b1):
    mol = mol_index.astype(_i32)
    pad = jnp.full((_NPAD - _NH,), _M, _i32)
    mol3a = jnp.concatenate([mol[:_NH], pad]).reshape(_NW, _GRP, _BATCH)
    mol3b = jnp.concatenate([mol[_NH:], pad]).reshape(_NW, _GRP, _BATCH)
    zeros = jnp.zeros((_M + 1, 8), _f32)
    dep = (jnp.asarray(n_molecules, _i32) - _M).astype(_f32)
    db = jnp.stack([jnp.full((16,), dep, _f32),
                    jnp.full((16,), b1[0].astype(_f32), _f32)])
    w0p = jnp.zeros((_D, 8), jnp.bfloat16).at[:, 0].set(
        W0[0].astype(jnp.bfloat16))
    w1p = jnp.zeros((_D, 8), jnp.bfloat16).at[:, 1].set(
        W1[0].astype(jnp.bfloat16))
    vals_a = _run_tc(all_features, w0p, w1p, 0)
    pa, atomen_a, ahier_a = _sc_segsum(vals_a, mol3a, db, zeros)
    vals_b = _run_tc(all_features, w0p, w1p, 1)
    pb, atomen_b, ahier_b = _sc_segsum(vals_b, mol3b, db, zeros)
    te, p0, p1, th, mh, bh = _run_fin(pa, pb)
    atomen = jnp.concatenate([atomen_a[:_NH], atomen_b[:_NH]]).reshape(_N, 1)
    ahier = jnp.concatenate([ahier_a[:_NH], ahier_b[:_NH]]).reshape(_N, 1)
    return (te, atomen, (p0, p1), th, ahier, mh, jnp.reshape(bh, ()))


# speedup vs baseline: 1.0065x; 1.0065x over previous
import functools

import jax
import jax.numpy as jnp
from jax.experimental import pallas as pl
from jax.experimental.pallas import tpu as pltpu
from jax.experimental.pallas import tpu_sc as plsc

_N = 160000
_D = 128
_M = 1024
_B = 8000
_NB = _N // _B

_NW = 32
_GRP = 40
_BATCH = 128
_CHUNK = _GRP * _BATCH
_NPAD = _NW * _CHUNK
_f32 = jnp.float32
_i32 = jnp.int32


def _tc_body(f0_ref, f1_ref, w0_ref, w1_ref, vals_ref):
    p0 = jax.lax.dot_general(
        f0_ref[0].astype(jnp.bfloat16), w0_ref[...],
        (((1,), (0,)), ((), ())), preferred_element_type=_f32)
    p1 = jax.lax.dot_general(
        f1_ref[0].astype(jnp.bfloat16), w1_ref[...],
        (((1,), (0,)), ((), ())), preferred_element_type=_f32)
    vals_ref[...] = p0 + p1


def _run_tc(feats, w0p, w1p):
    return pl.pallas_call(
        _tc_body,
        grid=(_NB,),
        in_specs=[
            pl.BlockSpec((1, _B, _D), lambda i: (0, i, 0)),
            pl.BlockSpec((1, _B, _D), lambda i: (1, i, 0)),
            pl.BlockSpec((_D, 8), lambda i: (0, 0)),
            pl.BlockSpec((_D, 8), lambda i: (0, 0)),
        ],
        out_specs=pl.BlockSpec((_B, 8), lambda i: (i, 0)),
        out_shape=jax.ShapeDtypeStruct((_NPAD, 8), _f32),
    )(feats, feats, w0p, w1p)


def _sc_body(vals_hbm, mol_hbm, db_hbm, zeros_hbm,
             part_hbm, atomen_hbm, ahier_hbm,
             idx_v, vals_v, atomen_v, ahier_v, db_v, acc_sh):
    c = jax.lax.axis_index("c")
    s = jax.lax.axis_index("s")
    wid = s * 2 + c
    base = wid * _CHUNK
    pltpu.sync_copy(mol_hbm.at[wid], idx_v)
    pltpu.sync_copy(vals_hbm.at[pl.ds(base, _CHUNK)], vals_v)
    pltpu.sync_copy(db_hbm, db_v)

    @pl.when(s == 0)
    def _zero():
        pltpu.sync_copy(zeros_hbm, acc_sh)

    dep = db_v[0]
    b1 = db_v[1]
    lanes = jax.lax.iota(_i32, 16)
    col0 = jnp.zeros((16,), _i32)

    def _atom(i, carry):
        rows = i * 16 + lanes
        pe0 = plsc.load_gather(vals_v, [rows, col0]) + dep
        pe1 = plsc.load_gather(vals_v, [rows, col0 + 1]) + b1
        e0s = pe0 * pe0
        e1s = pe1 * pe1
        den = e0s + e1s
        hier = e1s / den
        atomen_v[pl.ds(i * 16, 16)] = pe0 + pe1
        ahier_v[pl.ds(i * 16, 16)] = hier
        plsc.store_scatter(vals_v, [rows, col0], pe0)
        plsc.store_scatter(vals_v, [rows, col0 + 1], pe1)
        plsc.store_scatter(vals_v, [rows, col0 + 2], hier)
        plsc.store_scatter(vals_v, [rows, col0 + 3], e1s)
        plsc.store_scatter(vals_v, [rows, col0 + 4], den)
        return carry

    jax.lax.fori_loop(0, _CHUNK // 16, _atom, 0)
    pltpu.sync_copy(atomen_v, atomen_hbm.at[pl.ds(base, _CHUNK)])
    pltpu.sync_copy(ahier_v, ahier_hbm.at[pl.ds(base, _CHUNK)])
    plsc.subcore_barrier()

    def _grp(g, carry):
        pltpu.sync_copy(vals_v.at[pl.ds(g * _BATCH, _BATCH)],
                        acc_sh.at[idx_v.at[g]], add=True)
        return carry

    jax.lax.fori_loop(0, _GRP, _grp, 0)
    plsc.subcore_barrier()

    @pl.when(s == 0)
    def _flush():
        pltpu.sync_copy(acc_sh.at[pl.ds(0, _M)], part_hbm.at[c])


_sc_segsum = functools.partial(
    pl.kernel,
    out_type=[
        jax.ShapeDtypeStruct((2, _M, 8), _f32),
        jax.ShapeDtypeStruct((_NPAD,), _f32),
        jax.ShapeDtypeStruct((_NPAD,), _f32),
    ],
    mesh=plsc.VectorSubcoreMesh(core_axis_name="c", subcore_axis_name="s"),
    compiler_params=pltpu.CompilerParams(use_tc_tiling_on_sc=False,
                                         needs_layout_passes=False),
    scratch_types=[
        pltpu.VMEM((_GRP, _BATCH), _i32),
        pltpu.VMEM((_CHUNK, 8), _f32),
        pltpu.VMEM((_CHUNK,), _f32),
        pltpu.VMEM((_CHUNK,), _f32),
        pltpu.VMEM((2, 16), _f32),
        pltpu.VMEM_SHARED((_M + 1, 8), _f32),
    ],
)(_sc_body)


def _fin_body(part_ref, te_ref, p0_ref, p1_ref, th_ref, mh_ref, bh_ref):
    p = part_ref[0] + part_ref[1]
    t0 = p[:, 0:1]
    t1 = p[:, 1:2]
    te_ref[...] = t0 + t1
    p0_ref[...] = t0
    p1_ref[...] = t0 + t1
    th_ref[...] = p[:, 2:3]
    mh_ref[...] = p[:, 3:4] / p[:, 4:5]
    bh_ref[...] = (jnp.sum(p[:, 3:4], keepdims=True) /
                   jnp.sum(p[:, 4:5], keepdims=True))


def _run_fin(partials):
    m1 = [jax.ShapeDtypeStruct((_M, 1), _f32)] * 5
    return pl.pallas_call(
        _fin_body,
        out_shape=m1 + [jax.ShapeDtypeStruct((1, 1), _f32)],
    )(partials)


def kernel(all_features, mol_index, n_molecules, W0, W1, b1):
    mol = mol_index.astype(_i32)
    mol3 = jnp.concatenate(
        [mol, jnp.full((_NPAD - _N,), _M, _i32)]).reshape(_NW, _GRP, _BATCH)
    zeros = jnp.zeros((_M + 1, 8), _f32)
    dep = (jnp.asarray(n_molecules, _i32) - _M).astype(_f32)
    db = jnp.stack([jnp.full((16,), dep, _f32),
                    jnp.full((16,), b1[0].astype(_f32), _f32)])
    w0p = jnp.zeros((_D, 8), jnp.bfloat16).at[:, 0].set(
        W0[0].astype(jnp.bfloat16))
    w1p = jnp.zeros((_D, 8), jnp.bfloat16).at[:, 1].set(
        W1[0].astype(jnp.bfloat16))
    vals = _run_tc(all_features, w0p, w1p)
    partials, atomen, ahier = _sc_segsum(vals, mol3, db, zeros)
    te, p0, p1, th, mh, bh = _run_fin(partials)
    atomen = atomen[:_N].reshape(_N, 1)
    ahier = ahier[:_N].reshape(_N, 1)
    return (te, atomen, (p0, p1), th, ahier, mh, jnp.reshape(bh, ()))


# speedup vs baseline: 1.0158x; 1.0093x over previous
import functools

import jax
import jax.numpy as jnp
from jax.experimental import pallas as pl
from jax.experimental.pallas import tpu as pltpu
from jax.experimental.pallas import tpu_sc as plsc

_N = 160000
_D = 128
_M = 1024
_B = 8000
_NB = _N // _B

_NW = 32
_GRP = 40
_BATCH = 128
_CHUNK = _GRP * _BATCH
_NPAD = _NW * _CHUNK
_TAIL = _N - (_NW - 1) * _CHUNK
_f32 = jnp.float32
_i32 = jnp.int32


def _tc_body(f0_ref, f1_ref, w0_ref, w1_ref, vals_ref):
    p0 = jax.lax.dot_general(
        f0_ref[0].astype(jnp.bfloat16), w0_ref[...],
        (((1,), (0,)), ((), ())), preferred_element_type=_f32)
    p1 = jax.lax.dot_general(
        f1_ref[0].astype(jnp.bfloat16), w1_ref[...],
        (((1,), (0,)), ((), ())), preferred_element_type=_f32)
    vals_ref[...] = p0 + p1


def _run_tc(feats, w0p, w1p):
    return pl.pallas_call(
        _tc_body,
        grid=(_NB,),
        in_specs=[
            pl.BlockSpec((1, _B, _D), lambda i: (0, i, 0)),
            pl.BlockSpec((1, _B, _D), lambda i: (1, i, 0)),
            pl.BlockSpec((_D, 8), lambda i: (0, 0)),
            pl.BlockSpec((_D, 8), lambda i: (0, 0)),
        ],
        out_specs=pl.BlockSpec((_B, 8), lambda i: (i, 0)),
        out_shape=jax.ShapeDtypeStruct((_NPAD, 8), _f32),
    )(feats, feats, w0p, w1p)


def _sc_body(vals_hbm, mol_hbm, db_hbm, zeros_hbm,
             part_hbm, atomen_hbm, ahier_hbm,
             idx_v, vals_v, atomen_v, ahier_v, db_v, acc_sh):
    c = jax.lax.axis_index("c")
    s = jax.lax.axis_index("s")
    wid = s * 2 + c
    base = wid * _CHUNK
    pltpu.sync_copy(mol_hbm.at[wid], idx_v)
    pltpu.sync_copy(vals_hbm.at[pl.ds(base, _CHUNK)], vals_v)
    pltpu.sync_copy(db_hbm, db_v)

    @pl.when(s == 0)
    def _zero():
        pltpu.sync_copy(zeros_hbm, acc_sh)

    dep = db_v[0]
    b1 = db_v[1]
    lanes = jax.lax.iota(_i32, 16)
    col0 = jnp.zeros((16,), _i32)

    def _atom(i, carry):
        rows = i * 16 + lanes
        pe0 = plsc.load_gather(vals_v, [rows, col0]) + dep
        pe1 = plsc.load_gather(vals_v, [rows, col0 + 1]) + b1
        e0s = pe0 * pe0
        e1s = pe1 * pe1
        den = e0s + e1s
        hier = e1s / den
        atomen_v[pl.ds(i * 16, 16)] = pe0 + pe1
        ahier_v[pl.ds(i * 16, 16)] = hier
        plsc.store_scatter(vals_v, [rows, col0], pe0)
        plsc.store_scatter(vals_v, [rows, col0 + 1], pe1)
        plsc.store_scatter(vals_v, [rows, col0 + 2], hier)
        plsc.store_scatter(vals_v, [rows, col0 + 3], e1s)
        plsc.store_scatter(vals_v, [rows, col0 + 4], den)
        return carry

    jax.lax.fori_loop(0, _CHUNK // 16, _atom, 0)

    @pl.when(wid != _NW - 1)
    def _full_rows():
        pltpu.sync_copy(atomen_v, atomen_hbm.at[pl.ds(base, _CHUNK)])
        pltpu.sync_copy(ahier_v, ahier_hbm.at[pl.ds(base, _CHUNK)])

    @pl.when(wid == _NW - 1)
    def _tail_rows():
        pltpu.sync_copy(atomen_v.at[pl.ds(0, _TAIL)],
                        atomen_hbm.at[pl.ds(base, _TAIL)])
        pltpu.sync_copy(ahier_v.at[pl.ds(0, _TAIL)],
                        ahier_hbm.at[pl.ds(base, _TAIL)])

    plsc.subcore_barrier()

    def _grp(g, carry):
        pltpu.sync_copy(vals_v.at[pl.ds(g * _BATCH, _BATCH)],
                        acc_sh.at[idx_v.at[g]], add=True)
        return carry

    jax.lax.fori_loop(0, _GRP, _grp, 0)
    plsc.subcore_barrier()

    @pl.when(s == 0)
    def _flush():
        pltpu.sync_copy(acc_sh.at[pl.ds(0, _M)], part_hbm.at[c])


_sc_segsum = functools.partial(
    pl.kernel,
    out_type=[
        jax.ShapeDtypeStruct((2, _M, 8), _f32),
        jax.ShapeDtypeStruct((_N,), _f32),
        jax.ShapeDtypeStruct((_N,), _f32),
    ],
    mesh=plsc.VectorSubcoreMesh(core_axis_name="c", subcore_axis_name="s"),
    compiler_params=pltpu.CompilerParams(use_tc_tiling_on_sc=False,
                                         needs_layout_passes=False),
    scratch_types=[
        pltpu.VMEM((_GRP, _BATCH), _i32),
        pltpu.VMEM((_CHUNK, 8), _f32),
        pltpu.VMEM((_CHUNK,), _f32),
        pltpu.VMEM((_CHUNK,), _f32),
        pltpu.VMEM((2, 16), _f32),
        pltpu.VMEM_SHARED((_M + 1, 8), _f32),
    ],
)(_sc_body)


def _fin_body(part_ref, te_ref, p0_ref, p1_ref, th_ref, mh_ref, bh_ref):
    p = part_ref[0] + part_ref[1]
    t0 = p[:, 0:1]
    t1 = p[:, 1:2]
    te_ref[...] = t0 + t1
    p0_ref[...] = t0
    p1_ref[...] = t0 + t1
    th_ref[...] = p[:, 2:3]
    mh_ref[...] = p[:, 3:4] / p[:, 4:5]
    bh_ref[...] = (jnp.sum(p[:, 3:4], keepdims=True) /
                   jnp.sum(p[:, 4:5], keepdims=True))


def _run_fin(partials):
    m1 = [jax.ShapeDtypeStruct((_M, 1), _f32)] * 5
    return pl.pallas_call(
        _fin_body,
        out_shape=m1 + [jax.ShapeDtypeStruct((1, 1), _f32)],
    )(partials)


def kernel(all_features, mol_index, n_molecules, W0, W1, b1):
    mol = mol_index.astype(_i32)
    mol3 = jnp.concatenate(
        [mol, jnp.full((_NPAD - _N,), _M, _i32)]).reshape(_NW, _GRP, _BATCH)
    zeros = jnp.zeros((_M + 1, 8), _f32)
    dep = (jnp.asarray(n_molecules, _i32) - _M).astype(_f32)
    db = jnp.stack([jnp.full((16,), dep, _f32),
                    jnp.full((16,), b1[0].astype(_f32), _f32)])
    w0p = jnp.zeros((_D, 8), jnp.bfloat16).at[:, 0].set(
        W0[0].astype(jnp.bfloat16))
    w1p = jnp.zeros((_D, 8), jnp.bfloat16).at[:, 1].set(
        W1[0].astype(jnp.bfloat16))
    vals = _run_tc(all_features, w0p, w1p)
    partials, atomen, ahier = _sc_segsum(vals, mol3, db, zeros)
    te, p0, p1, th, mh, bh = _run_fin(partials)
    atomen = atomen.reshape(_N, 1)
    ahier = ahier.reshape(_N, 1)
    return (te, atomen, (p0, p1), th, ahier, mh, jnp.reshape(bh, ()))
